# baseline (speedup 1.0000x reference)
import jax
import jax.numpy as jnp
from jax.experimental import pallas as pl
from jax.experimental.pallas import tpu as pltpu

N_CODES = 1024
DIM = 256
HW = 1024
B = 16
BETA = 0.25


def _vq_body(z_ref, emb_ref, embs_ref, out_ref, idx_ref, loss_ref):
    b = pl.program_id(0)
    z = z_ref[0]
    emb = emb_ref[...]
    embs = embs_ref[...]
    mm = jax.lax.dot_general(embs, z, (((1,), (0,)), ((), ())),
                             preferred_element_type=jnp.float32)
    z2 = jnp.sum(z * z, axis=0, keepdims=True)
    e2 = jnp.sum(emb * emb, axis=1, keepdims=True)
    d = (z2 + e2) + mm
    m = jnp.min(d, axis=0, keepdims=True)
    row = jax.lax.broadcasted_iota(jnp.int32, d.shape, 0)
    idx = jnp.min(jnp.where(d == m, row, jnp.int32(2**30)), axis=0)
    idx_ref[0, 0, :] = idx
    onehot = (row == idx[None, :]).astype(jnp.float32)
    out_ref[0] = jax.lax.dot_general(emb, onehot, (((0,), (0,)), ((), ())),
                                     preferred_element_type=jnp.float32)

    @pl.when(b == 0)
    def _init():
        loss_ref[...] = jnp.zeros((1, 1), jnp.float32)

    loss_ref[...] += jnp.sum(m).reshape(1, 1)


def kernel(z, embedding):
    z3 = z.reshape(B, DIM, HW)
    out3, idx3, loss11 = pl.pallas_call(
        _vq_body,
        grid=(B,),
        in_specs=[
            pl.BlockSpec((1, DIM, HW), lambda b: (b, 0, 0)),
            pl.BlockSpec((N_CODES, DIM), lambda b: (0, 0)),
            pl.BlockSpec((N_CODES, DIM), lambda b: (0, 0)),
        ],
        out_specs=[
            pl.BlockSpec((1, DIM, HW), lambda b: (b, 0, 0)),
            pl.BlockSpec((1, 1, HW), lambda b: (b, 0, 0)),
            pl.BlockSpec((1, 1), lambda b: (0, 0)),
        ],
        out_shape=[
            jax.ShapeDtypeStruct((B, DIM, HW), jnp.float32),
            jax.ShapeDtypeStruct((B, 1, HW), jnp.int32),
            jax.ShapeDtypeStruct((1, 1), jnp.float32),
        ],
    )(z3, embedding, -2.0 * embedding)
    out = out3.reshape(z.shape)
    idx = idx3.reshape(-1)
    loss = loss11[0, 0] * (BETA / (B * HW * DIM))
    return out, loss, idx


# speedup vs baseline: 1.0303x; 1.0303x over previous
import jax
import jax.numpy as jnp
from jax.experimental import pallas as pl
from jax.experimental.pallas import tpu as pltpu

N_CODES = 1024
DIM = 256
HW = 1024
B = 16
BETA = 0.25


PAIR = 2


def _vq_body(z_ref, emb_ref, embs_ref, out_ref, idx_ref, loss_ref):
    b = pl.program_id(0)
    emb = emb_ref[...]
    embs = embs_ref[...]
    e2 = jnp.sum(emb * emb, axis=1, keepdims=True)
    step_loss = jnp.zeros((1, 1), jnp.float32)
    for j in range(PAIR):
        z = z_ref[j]
        mm = jax.lax.dot_general(embs, z, (((1,), (0,)), ((), ())),
                                 preferred_element_type=jnp.float32)
        z2 = jnp.sum(z * z, axis=0, keepdims=True)
        d = (z2 + e2) + mm
        m = jnp.min(d, axis=0, keepdims=True)
        row = jax.lax.broadcasted_iota(jnp.int32, d.shape, 0)
        idx = jnp.min(jnp.where(d == m, row, jnp.int32(2**30)), axis=0)
        idx_ref[j, 0, :] = idx
        onehot = (row == idx[None, :]).astype(jnp.float32)
        out_ref[j] = jax.lax.dot_general(emb, onehot, (((0,), (0,)), ((), ())),
                                         preferred_element_type=jnp.float32)
        step_loss = step_loss + jnp.sum(m).reshape(1, 1)

    @pl.when(b == 0)
    def _init():
        loss_ref[...] = jnp.zeros((1, 1), jnp.float32)

    loss_ref[...] += step_loss


def kernel(z, embedding):
    z3 = z.reshape(B, DIM, HW)
    out3, idx3, loss11 = pl.pallas_call(
        _vq_body,
        grid=(B // PAIR,),
        in_specs=[
            pl.BlockSpec((PAIR, DIM, HW), lambda b: (b, 0, 0)),
            pl.BlockSpec((N_CODES, DIM), lambda b: (0, 0)),
            pl.BlockSpec((N_CODES, DIM), lambda b: (0, 0)),
        ],
        out_specs=[
            pl.BlockSpec((PAIR, DIM, HW), lambda b: (b, 0, 0)),
            pl.BlockSpec((PAIR, 1, HW), lambda b: (b, 0, 0)),
            pl.BlockSpec((1, 1), lambda b: (0, 0)),
        ],
        out_shape=[
            jax.ShapeDtypeStruct((B, DIM, HW), jnp.float32),
            jax.ShapeDtypeStruct((B, 1, HW), jnp.int32),
            jax.ShapeDtypeStruct((1, 1), jnp.float32),
        ],
    )(z3, embedding, -2.0 * embedding)
    out = out3.reshape(z.shape)
    idx = idx3.reshape(-1)
    loss = loss11[0, 0] * (BETA / (B * HW * DIM))
    return out, loss, idx
